# Initial kernel scaffold; baseline (speedup 1.0000x reference)
import jax
import jax.numpy as jnp
from jax.experimental import pallas as pl


def kernel(input_ids, embed_w, bias):
    raise NotImplementedError("write your pallas kernel here")


# speedup vs baseline: 1.2037x; 1.2037x over previous
import functools

import jax
import jax.numpy as jnp
from jax import lax
from jax.experimental import pallas as pl
from jax.experimental.pallas import tpu as pltpu
from jax.experimental.pallas import tpu_sc as plsc

_BATCH = 16384
_N_FIELDS = 26
_NC = 2
_NS = 16
_NW = _NC * _NS

_ROWS_PER_TILE = _BATCH // _NW
_IDS_PER_TILE = _ROWS_PER_TILE * _N_FIELDS
_CHUNK = 128
_N_CHUNKS = _IDS_PER_TILE // _CHUNK
_FIRE = 8
_N_GROUPS = _N_CHUNKS // _FIRE
_L = 16


def _sc_body(ids_hbm, table_hbm, bias_hbm, out_hbm,
             idx_v, vals_v, out_v, bias_v, sem):
    wid = lax.axis_index("s") * _NC + lax.axis_index("c")

    pltpu.sync_copy(ids_hbm.at[wid], idx_v)
    pltpu.sync_copy(bias_hbm, bias_v.at[pl.ds(0, 1)])

    def gather_group(g, carry):
        copies = []
        for b in range(_FIRE):
            j = g * _FIRE + b
            copies.append(pltpu.async_copy(
                table_hbm.at[idx_v.at[j]],
                vals_v.at[pl.ds(j * _CHUNK, _CHUNK)],
                sem))
        for c in copies:
            c.wait()
        return carry

    lax.fori_loop(0, _N_GROUPS, gather_group, 0)

    bias_vec = plsc.load_gather(bias_v, [jnp.zeros((_L,), jnp.int32)])
    iota26 = lax.iota(jnp.int32, _L) * _N_FIELDS

    def reduce_chunk(c, carry):
        base = c * (_L * _N_FIELDS)
        acc = bias_vec
        for f in range(_N_FIELDS):
            g_idx = iota26 + (base + f)
            acc = acc + plsc.load_gather(vals_v, [g_idx])
        out_v[pl.ds(c * _L, _L)] = acc
        return carry

    lax.fori_loop(0, _ROWS_PER_TILE // _L, reduce_chunk, 0)

    pltpu.sync_copy(out_v, out_hbm.at[pl.ds(wid * _ROWS_PER_TILE, _ROWS_PER_TILE)])


@jax.jit
def _lr_sc(ids_blocked, table_flat, bias):
    mesh = plsc.VectorSubcoreMesh(core_axis_name="c", subcore_axis_name="s")
    kern = functools.partial(
        pl.kernel,
        mesh=mesh,
        out_type=jax.ShapeDtypeStruct((_BATCH,), jnp.float32),
        scratch_types=[
            pltpu.VMEM((_N_CHUNKS, _CHUNK), jnp.int32),
            pltpu.VMEM((_IDS_PER_TILE,), jnp.float32),
            pltpu.VMEM((_ROWS_PER_TILE,), jnp.float32),
            pltpu.VMEM((_L,), jnp.float32),
            pltpu.SemaphoreType.DMA,
        ],
        compiler_params=pltpu.CompilerParams(needs_layout_passes=False),
    )(_sc_body)
    return kern(ids_blocked, table_flat, bias)


def kernel(input_ids, embed_w, bias):
    ids_blocked = input_ids.reshape(_NW, _N_CHUNKS, _CHUNK)
    table_flat = embed_w.reshape(-1)
    out = _lr_sc(ids_blocked, table_flat, bias)
    return out.reshape(_BATCH, 1)


# speedup vs baseline: 1.3245x; 1.1003x over previous
import functools

import jax
import jax.numpy as jnp
from jax import lax
from jax.experimental import pallas as pl
from jax.experimental.pallas import tpu as pltpu
from jax.experimental.pallas import tpu_sc as plsc

_TABLE_ROWS = 1000000
_BATCH = 16384
_N_FIELDS = 26
_NC = 2
_NS = 16
_NW = _NC * _NS

_ROWS_PER_TILE = _BATCH // _NW
_CHUNK = 128
_CPF = _ROWS_PER_TILE // _CHUNK
_FIRE = 8
_N_DESC = _N_FIELDS * _CPF
_N_GROUPS = _N_DESC // _FIRE
_L = 16


def _sc_body(ids_hbm, table_hbm, bias_hbm, out_hbm,
             idx_v, vals_v, out_v, bias_v, sem):
    wid = lax.axis_index("s") * _NC + lax.axis_index("c")
    base = wid * _ROWS_PER_TILE

    pltpu.sync_copy(ids_hbm.at[wid], idx_v)
    pltpu.sync_copy(bias_hbm, bias_v.at[pl.ds(0, 1)])

    def gather_group(g, carry):
        copies = []
        for b in range(_FIRE):
            j = g * _FIRE + b
            copies.append(pltpu.async_copy(
                table_hbm.at[idx_v.at[j]],
                vals_v.at[pl.ds(j * _CHUNK, _CHUNK)],
                sem))
        for cp in copies:
            cp.wait()
        return carry

    lax.fori_loop(0, _N_GROUPS, gather_group, 0)

    bias_vec = plsc.load_gather(bias_v, [jnp.zeros((_L,), jnp.int32)])

    iota16 = lax.iota(jnp.int32, _L)

    def reduce_chunk(c, carry):
        rbase = c * _L
        acc = bias_vec
        for f in range(_N_FIELDS):
            acc = acc + plsc.load_gather(vals_v, [iota16 + (rbase + f * _ROWS_PER_TILE)])
        out_v[pl.ds(rbase, _L)] = acc
        return carry

    lax.fori_loop(0, _ROWS_PER_TILE // _L, reduce_chunk, 0)

    pltpu.sync_copy(out_v, out_hbm.at[pl.ds(base, _ROWS_PER_TILE)])


@jax.jit
def _lr_sc(ids_t, table_flat, bias):
    mesh = plsc.VectorSubcoreMesh(core_axis_name="c", subcore_axis_name="s")
    kern = functools.partial(
        pl.kernel,
        mesh=mesh,
        out_type=jax.ShapeDtypeStruct((_BATCH,), jnp.float32),
        scratch_types=[
            pltpu.VMEM((_N_DESC, _CHUNK), jnp.int32),
            pltpu.VMEM((_N_FIELDS * _ROWS_PER_TILE,), jnp.float32),
            pltpu.VMEM((_ROWS_PER_TILE,), jnp.float32),
            pltpu.VMEM((_L,), jnp.float32),
            pltpu.SemaphoreType.DMA,
        ],
        compiler_params=pltpu.CompilerParams(needs_layout_passes=False),
    )(_sc_body)
    return kern(ids_t, table_flat, bias)


def kernel(input_ids, embed_w, bias):
    ids_fm = (input_ids.T
              .reshape(_N_FIELDS, _NW, _ROWS_PER_TILE)
              .swapaxes(0, 1)
              .reshape(_NW, _N_DESC, _CHUNK))
    table_flat = embed_w.reshape(-1)
    out = _lr_sc(ids_fm, table_flat, bias)
    return out.reshape(_BATCH, 1)


# speedup vs baseline: 2.1162x; 1.5978x over previous
import functools

import jax
import jax.numpy as jnp
from jax import lax
from jax.experimental import pallas as pl
from jax.experimental.pallas import tpu as pltpu
from jax.experimental.pallas import tpu_sc as plsc

_TABLE_ROWS = 1000000
_TABLE_PAD = 1000448
_BATCH = 16384
_N_FIELDS = 26
_NC = 2
_NS = 16
_NW = _NC * _NS

_ROWS_PER_TILE = _BATCH // _NW
_CHUNK = 128
_CPF = _ROWS_PER_TILE // _CHUNK
_FIRE = 8
_N_DESC = _N_FIELDS * _CPF
_N_GROUPS = _N_DESC // _FIRE
_L = 16


def _sc_body(ids_hbm, table_hbm, bias_hbm, out_hbm,
             idx_v, vals_v, out_v, bias_v, sem):
    wid = lax.axis_index("s") * _NC + lax.axis_index("c")
    base = wid * _ROWS_PER_TILE

    pltpu.sync_copy(ids_hbm.at[wid], idx_v)
    pltpu.sync_copy(bias_hbm, bias_v.at[pl.ds(0, 1)])

    def gather_group(g, carry):
        copies = []
        for b in range(_FIRE):
            j = g * _FIRE + b
            copies.append(pltpu.async_copy(
                table_hbm.at[idx_v.at[j]],
                vals_v.at[pl.ds(j * _CHUNK, _CHUNK)],
                sem))
        for cp in copies:
            cp.wait()
        return carry

    lax.fori_loop(0, _N_GROUPS, gather_group, 0)

    bias_vec = plsc.load_gather(bias_v, [jnp.zeros((_L,), jnp.int32)])

    iota16 = lax.iota(jnp.int32, _L)

    def reduce_chunk(c, carry):
        rbase = c * _L
        acc = bias_vec
        for f in range(_N_FIELDS):
            acc = acc + plsc.load_gather(vals_v, [iota16 + (rbase + f * _ROWS_PER_TILE)])
        out_v[pl.ds(rbase, _L)] = acc
        return carry

    lax.fori_loop(0, _ROWS_PER_TILE // _L, reduce_chunk, 0)

    pltpu.sync_copy(out_v, out_hbm.at[pl.ds(base, _ROWS_PER_TILE)])


@jax.jit
def _lr_sc(ids_t, table_flat, bias):
    mesh = plsc.VectorSubcoreMesh(core_axis_name="c", subcore_axis_name="s")
    kern = functools.partial(
        pl.kernel,
        mesh=mesh,
        out_type=jax.ShapeDtypeStruct((_BATCH,), jnp.float32),
        scratch_types=[
            pltpu.VMEM((_N_DESC, _CHUNK), jnp.int32),
            pltpu.VMEM((_N_FIELDS * _ROWS_PER_TILE,), jnp.float32),
            pltpu.VMEM((_ROWS_PER_TILE,), jnp.float32),
            pltpu.VMEM((_L,), jnp.float32),
            pltpu.SemaphoreType.DMA,
        ],
        compiler_params=pltpu.CompilerParams(needs_layout_passes=False),
    )(_sc_body)
    return kern(ids_t, table_flat, bias)


def kernel(input_ids, embed_w, bias):
    ids_fm = (input_ids.T
              .reshape(_N_FIELDS, _NW, _ROWS_PER_TILE)
              .swapaxes(0, 1)
              .reshape(_NW, _N_DESC, _CHUNK))
    padded = jnp.pad(embed_w, ((0, _TABLE_PAD - _TABLE_ROWS), (0, 0)))
    table_flat = padded.reshape(_TABLE_PAD)
    out = _lr_sc(ids_fm, table_flat, bias)
    return out.reshape(_BATCH, 1)


# speedup vs baseline: 2.1167x; 1.0002x over previous
import functools

import jax
import jax.numpy as jnp
from jax import lax
from jax.experimental import pallas as pl
from jax.experimental.pallas import tpu as pltpu
from jax.experimental.pallas import tpu_sc as plsc

_TABLE_ROWS = 1000000
_TABLE_PAD = 1000448
_BATCH = 16384
_N_FIELDS = 26
_NC = 2
_NS = 16
_NW = _NC * _NS

_ROWS_PER_TILE = _BATCH // _NW
_CHUNK = 128
_CPF = _ROWS_PER_TILE // _CHUNK
_FIRE = 8
_N_DESC = _N_FIELDS * _CPF
_N_GROUPS = _N_DESC // _FIRE
_L = 16


def _sc_body(ids_hbm, table_hbm, bias_hbm, out_hbm,
             idx_v, vals_v, out_v, bias_v, sem):
    wid = lax.axis_index("s") * _NC + lax.axis_index("c")
    base = wid * _ROWS_PER_TILE

    pltpu.sync_copy(ids_hbm.at[wid], idx_v)
    pltpu.sync_copy(bias_hbm, bias_v.at[pl.ds(0, 1)])

    def gather_group(g, carry):
        copies = []
        for b in range(_FIRE):
            j = g * _FIRE + b
            copies.append(pltpu.async_copy(
                table_hbm.at[idx_v.at[j]],
                vals_v.at[pl.ds(j * _CHUNK, _CHUNK)],
                sem))
        for cp in copies:
            cp.wait()
        return carry

    lax.fori_loop(0, _N_GROUPS, gather_group, 0)

    bias_vec = plsc.load_gather(bias_v, [jnp.zeros((_L,), jnp.int32)])

    iota16 = lax.iota(jnp.int32, _L)

    def reduce_chunk(c, carry):
        rbase = c * _L
        acc = bias_vec
        for f in range(_N_FIELDS):
            acc = acc + plsc.load_gather(vals_v, [iota16 + (rbase + f * _ROWS_PER_TILE)])
        out_v[pl.ds(rbase, _L)] = acc
        return carry

    lax.fori_loop(0, _ROWS_PER_TILE // _L, reduce_chunk, 0)

    pltpu.sync_copy(out_v, out_hbm.at[pl.ds(base, _ROWS_PER_TILE)])


@jax.jit
def _lr_sc(ids_t, table_flat, bias):
    mesh = plsc.VectorSubcoreMesh(core_axis_name="c", subcore_axis_name="s")
    kern = functools.partial(
        pl.kernel,
        mesh=mesh,
        out_type=jax.ShapeDtypeStruct((_BATCH,), jnp.float32),
        scratch_types=[
            pltpu.VMEM((_N_DESC, _CHUNK), jnp.int32),
            pltpu.VMEM((_N_FIELDS * _ROWS_PER_TILE,), jnp.float32),
            pltpu.VMEM((_ROWS_PER_TILE,), jnp.float32),
            pltpu.VMEM((_L,), jnp.float32),
            pltpu.SemaphoreType.DMA,
        ],
        compiler_params=pltpu.CompilerParams(needs_layout_passes=False),
    )(_sc_body)
    return kern(ids_t, table_flat, bias)


def kernel(input_ids, embed_w, bias):
    ids_fm = (input_ids.T
              .reshape(_N_FIELDS, _NW, _ROWS_PER_TILE)
              .swapaxes(0, 1)
              .reshape(_NW, _N_DESC, _CHUNK))
    padded = jnp.pad(embed_w, ((0, _TABLE_PAD - _TABLE_ROWS), (0, 0)))
    flat_pad = jax.lax.optimization_barrier(padded.reshape(_TABLE_PAD))
    table_flat = flat_pad[:_TABLE_ROWS]
    out = _lr_sc(ids_fm, table_flat, bias)
    return out.reshape(_BATCH, 1)
